# Initial kernel scaffold; baseline (speedup 1.0000x reference)
import functools

import jax
import jax.numpy as jnp
from jax import lax
from jax.experimental import pallas as pl
from jax.experimental.pallas import tpu as pltpu
from jax.experimental.pallas import tpu_sc as plsc

_H = 16
_S = 4096
_NC = 2
_NS = 16
_NW = _NC * _NS
_ROWS_PER_W = _H * _S // _NW
_QPR = _ROWS_PER_W // 8


def _sc_body(shifts_hbm, out_hbm, buf, sem):
    wid = lax.axis_index("s") * _NC + lax.axis_index("c")
    h = wid // 2
    half = wid % 2
    pltpu.sync_copy(shifts_hbm.at[h], buf)

    q0 = (1 - half) * _QPR
    row_base = h * _S

    for r in range(8):
        def body(qi, _, r=r):
            q = q0 + qi
            src = buf.at[pl.ds(r, 1), pl.ds(pl.multiple_of(8 * q, 8), _S)]
            m = row_base + (_S - 1) - 8 * q - r
            pltpu.async_copy(src, out_hbm.at[pl.ds(m, 1), :], sem)
            return 0
        lax.fori_loop(0, _QPR, body, 0)

    my_rows = out_hbm.at[pl.ds(row_base + half * _ROWS_PER_W, _ROWS_PER_W), :]
    pltpu.make_async_copy(my_rows, my_rows, sem).wait()


@jax.jit
def _expand(shifts):
    mesh = plsc.VectorSubcoreMesh(core_axis_name="c", subcore_axis_name="s")
    return pl.kernel(
        _sc_body,
        out_type=jax.ShapeDtypeStruct((_H * _S, _S), jnp.float32),
        mesh=mesh,
        scratch_types=[
            pltpu.VMEM((8, 8192), jnp.float32),
            pltpu.SemaphoreType.DMA,
        ],
    )(shifts)


def kernel(qlen, klen, relative_attention_bias):
    tt = relative_attention_bias.T
    ttp = jnp.pad(tt, ((0, 0), (0, 8192 + 7 - tt.shape[1])))
    shifts = jnp.stack([ttp[:, r:r + 8192] for r in range(8)], axis=1)
    out = _expand(shifts)
    return out.reshape(1, _H, _S, _S)


# speedup vs baseline: 42.3373x; 42.3373x over previous
import functools

import jax
import jax.numpy as jnp
from jax import lax
from jax.experimental import pallas as pl
from jax.experimental.pallas import tpu as pltpu
from jax.experimental.pallas import tpu_sc as plsc

_H = 16
_S = 4096
_NC = 2
_NS = 16
_NW = _NC * _NS
_ROWS_PER_W = _H * _S // _NW
_QPR = _ROWS_PER_W // 8


def _sc_body(shifts_hbm, out_hbm, buf, sem):
    wid = lax.axis_index("s") * _NC + lax.axis_index("c")
    h = wid // 2
    half = wid % 2
    pltpu.sync_copy(shifts_hbm.at[h], buf)

    q0 = (1 - half) * _QPR
    row_base = h * _S

    for r in range(8):
        def body(qi, _, r=r):
            q = q0 + qi
            src = buf.at[pl.ds(pl.multiple_of(r * 8192 + 8 * q, 8), _S)]
            m = row_base + (_S - 1) - 8 * q - r
            dst = out_hbm.at[pl.ds(pl.multiple_of(m * _S, 8), _S)]
            pltpu.sync_copy(src, dst)
            return 0
        lax.fori_loop(0, _QPR, body, 0)


@jax.jit
def _expand(shifts):
    mesh = plsc.VectorSubcoreMesh(core_axis_name="c", subcore_axis_name="s")
    return pl.kernel(
        _sc_body,
        out_type=jax.ShapeDtypeStruct((_H * _S * _S,), jnp.float32),
        mesh=mesh,
        scratch_types=[
            pltpu.VMEM((8 * 8192,), jnp.float32),
            pltpu.SemaphoreType.DMA,
        ],
    )(shifts)


def kernel(qlen, klen, relative_attention_bias):
    tt = relative_attention_bias.T
    ttp = jnp.pad(tt, ((0, 0), (0, 8192 + 7 - tt.shape[1])))
    shifts = jnp.stack([ttp[:, r:r + 8192] for r in range(8)], axis=1)
    out = _expand(shifts.reshape(_H, 8 * 8192))
    return out.reshape(1, _H, _S, _S)


# speedup vs baseline: 42.4440x; 1.0025x over previous
import functools

import jax
import jax.numpy as jnp
from jax import lax
from jax.experimental import pallas as pl
from jax.experimental.pallas import tpu as pltpu
from jax.experimental.pallas import tpu_sc as plsc

_H = 16
_S = 4096
_NC = 2
_NS = 16
_NW = _NC * _NS
_ROWS_PER_W = _H * _S // _NW
_QPR = _ROWS_PER_W // 8


def _sc_body(shifts_hbm, out_hbm, buf, sem):
    wid = lax.axis_index("s") * _NC + lax.axis_index("c")
    h = wid // 2
    half = wid % 2
    pltpu.sync_copy(shifts_hbm.at[h], buf)

    q0 = (1 - half) * _QPR
    row_base = h * _S

    for r in range(8):
        @plsc.parallel_loop(0, _QPR, unroll=4)
        def _(qi, r=r):
            q = q0 + qi
            src = buf.at[pl.ds(pl.multiple_of(r * 8192 + 8 * q, 8), _S)]
            m = row_base + (_S - 1) - 8 * q - r
            dst = out_hbm.at[pl.ds(pl.multiple_of(m * _S, 8), _S)]
            pltpu.sync_copy(src, dst)


@jax.jit
def _expand(shifts):
    mesh = plsc.VectorSubcoreMesh(core_axis_name="c", subcore_axis_name="s")
    return pl.kernel(
        _sc_body,
        out_type=jax.ShapeDtypeStruct((_H * _S * _S,), jnp.float32),
        mesh=mesh,
        scratch_types=[
            pltpu.VMEM((8 * 8192,), jnp.float32),
            pltpu.SemaphoreType.DMA,
        ],
    )(shifts)


def kernel(qlen, klen, relative_attention_bias):
    tt = relative_attention_bias.T
    ttp = jnp.pad(tt, ((0, 0), (0, 8192 + 7 - tt.shape[1])))
    shifts = jnp.stack([ttp[:, r:r + 8192] for r in range(8)], axis=1)
    out = _expand(shifts.reshape(_H, 8 * 8192))
    return out.reshape(1, _H, _S, _S)


# speedup vs baseline: 45.2650x; 1.0665x over previous
import functools

import jax
import jax.numpy as jnp
from jax import lax
from jax.experimental import pallas as pl
from jax.experimental.pallas import tpu as pltpu
from jax.experimental.pallas import tpu_sc as plsc

_H = 16
_S = 4096
_NC = 2
_NS = 16
_NW = _NC * _NS
_ROWS_PER_W = _H * _S // _NW
_QPR = _ROWS_PER_W // 8


def _sc_body(shifts_hbm, out_hbm, buf, sem):
    wid = lax.axis_index("s") * _NC + lax.axis_index("c")
    h = wid // 2
    half = wid % 2
    pltpu.sync_copy(shifts_hbm.at[h], buf)

    q0 = (1 - half) * _QPR
    row_base = h * _S

    lag = 8
    drain_one = pltpu.make_async_copy(
        buf.at[pl.ds(0, _S)], out_hbm.at[pl.ds(0, _S)], sem)

    def body(t, _):
        r = t % 8
        q = q0 + t // 8
        src = buf.at[pl.ds(pl.multiple_of(r * 8192 + 8 * q, 8), _S)]
        m = row_base + (_S - 1) - 8 * q - r
        dst = out_hbm.at[pl.ds(pl.multiple_of(m * _S, 8), _S)]
        pltpu.async_copy(src, dst, sem)

        @pl.when(t >= lag)
        def _():
            drain_one.wait()
        return 0

    lax.fori_loop(0, _ROWS_PER_W, body, 0)
    for _ in range(lag):
        drain_one.wait()


@jax.jit
def _expand(shifts):
    mesh = plsc.VectorSubcoreMesh(core_axis_name="c", subcore_axis_name="s")
    return pl.kernel(
        _sc_body,
        out_type=jax.ShapeDtypeStruct((_H * _S * _S,), jnp.float32),
        mesh=mesh,
        scratch_types=[
            pltpu.VMEM((8 * 8192,), jnp.float32),
            pltpu.SemaphoreType.DMA,
        ],
    )(shifts)


def kernel(qlen, klen, relative_attention_bias):
    tt = relative_attention_bias.T
    ttp = jnp.pad(tt, ((0, 0), (0, 8192 + 7 - tt.shape[1])))
    shifts = jnp.stack([ttp[:, r:r + 8192] for r in range(8)], axis=1)
    out = _expand(shifts.reshape(_H, 8 * 8192))
    return out.reshape(1, _H, _S, _S)
